# baseline (speedup 1.0000x reference)
import functools

import jax
import jax.numpy as jnp
from jax import lax
from jax.experimental import pallas as pl
from jax.experimental.pallas import tpu as pltpu
from jax.experimental.pallas import tpu_sc as plsc

_DIM = 32
_VPAD = 128
_C = 128
_K = 8
_NC = 2
_NS = 16
_NW = _NC * _NS
_BROWS = _VPAD // _NS


def _vbroadcast(vec, idx16):
    return lax.gather(
        vec, idx16[:, None],
        lax.GatherDimensionNumbers(
            offset_dims=(), collapsed_slice_dims=(0,), start_index_map=(0,)),
        (1,), mode=lax.GatherScatterMode.PROMISE_IN_BOUNDS)


@functools.cache
def _make_sc_kernel(n_rows):
    rows_per_w = n_rows // _NW
    steps = rows_per_w // _K
    mesh = plsc.VectorSubcoreMesh(core_axis_name="c", subcore_axis_name="s")

    @functools.partial(
        pl.kernel,
        mesh=mesh,
        compiler_params=pltpu.CompilerParams(
            needs_layout_passes=False, use_tc_tiling_on_sc=False),
        out_type=jax.ShapeDtypeStruct((n_rows, _C, _DIM), jnp.float32),
        scratch_types=[
            pltpu.VMEM((_VPAD // 4, 128), jnp.float32),
            pltpu.VMEM((_DIM * _DIM // 128, 128), jnp.float32),
            pltpu.VMEM((128,), jnp.float32),
            pltpu.VMEM((_VPAD, _DIM), jnp.float32),
            pltpu.VMEM((2, _K, _C), jnp.int32),
            pltpu.VMEM((2, _K, _C, _DIM), jnp.float32),
            pltpu.SemaphoreType.DMA,
            pltpu.SemaphoreType.DMA,
        ],
    )
    def sc_kernel(idx_hbm, e_hbm, w_hbm, b_hbm, out_hbm,
                  e_vs, w_v, b_v, t2d, idx_v, out_v, sem_i, sem_o):
        sid = lax.axis_index("s")
        pltpu.sync_copy(e_hbm, e_vs)
        pltpu.sync_copy(w_hbm, w_v)
        pltpu.sync_copy(b_hbm, b_v)
        b0 = b_v[pl.ds(0, 16)]
        b1 = b_v[pl.ds(16, 16)]

        def build_row(v, carry):
            r = v >> 2
            c = (v & 3) * 32
            rowv = jnp.full((16,), r, dtype=jnp.int32)
            acc0, acc1 = b0, b1
            for k in range(_DIM):
                ek = plsc.load_gather(
                    e_vs, [rowv, jnp.full((16,), c + k, dtype=jnp.int32)])
                wf = k * _DIM
                w0 = w_v[wf // 128, pl.ds(wf % 128, 16)]
                w1 = w_v[wf // 128, pl.ds(wf % 128 + 16, 16)]
                acc0 = acc0 + ek * w0
                acc1 = acc1 + ek * w1
            t2d[v, pl.ds(0, 16)] = acc0
            t2d[v, pl.ds(16, 16)] = acc1
            return carry

        lax.fori_loop(0, _VPAD, build_row, 0)

        wid = sid * _NC + lax.axis_index("c")
        row0 = wid * rows_per_w

        def fire_idx(s, p):
            r = row0 + s * _K
            pltpu.async_copy(idx_hbm.at[pl.ds(r, _K)], idx_v.at[p], sem_i)

        def wait_idx(p):
            pltpu.make_async_copy(
                idx_hbm.at[pl.ds(row0, _K)], idx_v.at[p], sem_i).wait()

        def wait_out():
            pltpu.make_async_copy(
                out_v.at[0], out_hbm.at[pl.ds(row0, _K)], sem_o).wait()

        fire_idx(0, 0)
        cols0 = lax.iota(jnp.int32, 16)
        lsel = [jnp.full((16,), l, dtype=jnp.int32) for l in range(16)]

        def outer(o, carry):
            for p in range(2):
                s = o * 2 + p
                wait_idx(p)

                @pl.when(s + 1 < steps)
                def _prefetch():
                    fire_idx(s + 1, 1 - p)

                @pl.when(s >= 2)
                def _reclaim():
                    wait_out()

                ob = out_v.at[p]

                def grp(i, c2):
                    iv = idx_v[p, i >> 3, pl.ds((i & 7) * 16, 16)]
                    d0 = i >> 3
                    d1 = (i & 7) * 16
                    for l in range(16):
                        bvl = _vbroadcast(iv, lsel[l])
                        g0 = plsc.load_gather(t2d, [bvl, cols0])
                        g1 = plsc.load_gather(t2d, [bvl, cols0 + 16])
                        ob[d0, d1 + l, pl.ds(0, 16)] = g0
                        ob[d0, d1 + l, pl.ds(16, 16)] = g1
                    return c2

                lax.fori_loop(0, _K * (_C // 16), grp, 0)
                pltpu.async_copy(
                    ob, out_hbm.at[pl.ds(row0 + s * _K, _K)], sem_o)
            return carry

        lax.fori_loop(0, steps // 2, outer, 0)
        wait_out()
        wait_out()

    return sc_kernel


def kernel(x, embed_table, W, b):
    bsz, hist = x.shape
    n = bsz * hist
    idx = x.reshape(n // _C, _C).astype(jnp.int32)
    e2 = jnp.pad(embed_table.astype(jnp.float32),
                 ((0, _VPAD - embed_table.shape[0]), (0, 0))).reshape(-1, 128)
    w2 = W.astype(jnp.float32).T.reshape(-1, 128)
    b2 = jnp.pad(b.astype(jnp.float32), (0, 128 - _DIM))
    out = _make_sc_kernel(n // _C)(idx, e2, w2, b2)
    return out.reshape(bsz, hist, _DIM)


# speedup vs baseline: 7.0941x; 7.0941x over previous
import functools

import jax
import jax.numpy as jnp
from jax import lax
from jax.experimental import pallas as pl
from jax.experimental.pallas import tpu as pltpu
from jax.experimental.pallas import tpu_sc as plsc

_DIM = 32
_VPAD = 128
_NC = 2
_NS = 16
_NW = _NC * _NS
_BT = 512


@functools.cache
def _make_sc_kernel(npos, nbatch):
    steps = npos
    mesh = plsc.VectorSubcoreMesh(core_axis_name="c", subcore_axis_name="s")

    @functools.partial(
        pl.kernel,
        mesh=mesh,
        compiler_params=pltpu.CompilerParams(
            needs_layout_passes=False, use_tc_tiling_on_sc=True),
        out_type=jax.ShapeDtypeStruct((npos, _DIM, nbatch), jnp.float32),
        scratch_types=[
            pltpu.VMEM((_VPAD // 4, 128), jnp.float32),
            pltpu.VMEM((_DIM * _DIM // 128, 128), jnp.float32),
            pltpu.VMEM((128,), jnp.float32),
            pltpu.VMEM((_VPAD // 4, 128), jnp.float32),
            pltpu.VMEM((_DIM, _VPAD), jnp.float32),
            pltpu.VMEM((2, _BT), jnp.int32),
            pltpu.VMEM((2, _DIM, _BT), jnp.float32),
            pltpu.SemaphoreType.DMA,
            pltpu.SemaphoreType.DMA,
        ],
    )
    def sc_kernel(idx_hbm, e_hbm, w_hbm, b_hbm, out_hbm,
                  e_v, w_v, b_v, t_f, t_t, idx_v, st, sem_i, sem_o):
        pltpu.sync_copy(e_hbm, e_v)
        pltpu.sync_copy(w_hbm, w_v)
        pltpu.sync_copy(b_hbm, b_v)
        b0 = b_v[pl.ds(0, 16)]
        b1 = b_v[pl.ds(16, 16)]

        def build_row(v, carry):
            r = v >> 2
            c = (v & 3) * 32
            rowv = jnp.full((16,), r, dtype=jnp.int32)
            acc0, acc1 = b0, b1
            for k in range(_DIM):
                ek = plsc.load_gather(
                    e_v, [rowv, jnp.full((16,), c + k, dtype=jnp.int32)])
                wf = k * _DIM
                w0 = w_v[wf // 128, pl.ds(wf % 128, 16)]
                w1 = w_v[wf // 128, pl.ds(wf % 128 + 16, 16)]
                acc0 = acc0 + ek * w0
                acc1 = acc1 + ek * w1
            t_f[r, pl.ds(c, 16)] = acc0
            t_f[r, pl.ds(c + 16, 16)] = acc1
            return carry

        lax.fori_loop(0, _VPAD, build_row, 0)

        lane = lax.iota(jnp.int32, 16)
        for j in range(_DIM):
            for vb in range(_VPAD // 16):
                a = (lane + vb * 16) * _DIM + j
                g = plsc.load_gather(t_f, [a >> 7, a & 127])
                t_t[j, pl.ds(vb * 16, 16)] = g

        wid = lax.axis_index("s") * _NC + lax.axis_index("c")
        col0 = wid * _BT

        def fire_idx(l, p):
            pltpu.async_copy(
                idx_hbm.at[l, pl.ds(col0, _BT)], idx_v.at[p], sem_i)

        def wait_idx(p):
            pltpu.make_async_copy(
                idx_hbm.at[0, pl.ds(col0, _BT)], idx_v.at[p], sem_i).wait()

        def wait_out():
            pltpu.make_async_copy(
                st.at[0], out_hbm.at[0, :, pl.ds(col0, _BT)], sem_o).wait()

        fire_idx(0, 0)

        def outer(o, carry):
            for p in range(2):
                l = o * 2 + p
                wait_idx(p)

                @pl.when(l + 1 < steps)
                def _prefetch():
                    fire_idx(l + 1, 1 - p)

                @pl.when(l >= 2)
                def _reclaim():
                    wait_out()

                sp = st.at[p]

                @plsc.parallel_loop(0, _BT // 16, unroll=2)
                def _gather(bb):
                    b0 = bb * 16
                    iv = idx_v[p, pl.ds(b0, 16)]
                    for j in range(_DIM):
                        g = plsc.load_gather(
                            t_t, [jnp.full((16,), j, dtype=jnp.int32), iv])
                        sp[j, pl.ds(b0, 16)] = g

                pltpu.async_copy(
                    sp, out_hbm.at[l, :, pl.ds(col0, _BT)], sem_o)
            return carry

        lax.fori_loop(0, steps // 2, outer, 0)
        wait_out()
        wait_out()

    return sc_kernel


def kernel(x, embed_table, W, b):
    bsz, hist = x.shape
    xt = x.T.astype(jnp.int32)
    e2 = jnp.pad(embed_table.astype(jnp.float32),
                 ((0, _VPAD - embed_table.shape[0]), (0, 0))).reshape(-1, 128)
    w2 = W.astype(jnp.float32).T.reshape(-1, 128)
    b2 = jnp.pad(b.astype(jnp.float32), (0, 128 - _DIM))
    out3 = _make_sc_kernel(hist, bsz)(xt, e2, w2, b2)
    return out3.transpose(2, 0, 1)
